# baseline (speedup 1.0000x reference)
import functools

import jax
import jax.numpy as jnp
from jax import lax
from jax.experimental import pallas as pl
from jax.experimental.pallas import tpu as pltpu
from jax.experimental.pallas import tpu_sc as plsc

_NC = 2
_NS = 16
_NW = _NC * _NS
_CHUNK = 112
_DEPTH = 4


def _sc_gather_pair(ids_flat, elem_weight, hnet_weight):
    n = ids_flat.shape[0]
    emb = elem_weight.shape[1]
    per_w = n // _NW
    n_chunks = per_w // _CHUNK
    assert n_chunks % _DEPTH == 0
    mesh = plsc.VectorSubcoreMesh(core_axis_name="c", subcore_axis_name="s")

    buf_types = []
    for _ in range(_DEPTH):
        buf_types.append(pltpu.VMEM((_CHUNK, emb), jnp.float32))
        buf_types.append(pltpu.VMEM((_CHUNK, emb), jnp.float32))
        buf_types.append(pltpu.SemaphoreType.DMA)
        buf_types.append(pltpu.SemaphoreType.DMA)
        buf_types.append(pltpu.SemaphoreType.DMA)
        buf_types.append(pltpu.SemaphoreType.DMA)

    @functools.partial(
        pl.kernel,
        out_type=(
            jax.ShapeDtypeStruct((n, emb), jnp.float32),
            jax.ShapeDtypeStruct((n, emb), jnp.float32),
        ),
        mesh=mesh,
        scratch_types=[pltpu.VMEM((per_w,), jnp.int32)] + buf_types,
        compiler_params=pltpu.CompilerParams(use_tc_tiling_on_sc=True),
    )
    def sc_gather(ids_hbm, elem_hbm, hnet_hbm, out_e, out_h, idx_all, *bufs):
        ebuf = [bufs[6 * s] for s in range(_DEPTH)]
        hbuf = [bufs[6 * s + 1] for s in range(_DEPTH)]
        sem_ge = [bufs[6 * s + 2] for s in range(_DEPTH)]
        sem_gh = [bufs[6 * s + 3] for s in range(_DEPTH)]
        sem_we = [bufs[6 * s + 4] for s in range(_DEPTH)]
        sem_wh = [bufs[6 * s + 5] for s in range(_DEPTH)]

        wid = lax.axis_index("s") * _NC + lax.axis_index("c")
        base = wid * per_w
        pltpu.sync_copy(ids_hbm.at[pl.ds(base, per_w)], idx_all)

        def g_copies(s, chunk):
            isl = idx_all.at[pl.ds(chunk * _CHUNK, _CHUNK)]
            return (
                pltpu.make_async_copy(elem_hbm.at[isl], ebuf[s], sem_ge[s]),
                pltpu.make_async_copy(hnet_hbm.at[isl], hbuf[s], sem_gh[s]),
            )

        def w_copies(s, chunk):
            off = base + chunk * _CHUNK
            return (
                pltpu.make_async_copy(
                    ebuf[s], out_e.at[pl.ds(off, _CHUNK)], sem_we[s]),
                pltpu.make_async_copy(
                    hbuf[s], out_h.at[pl.ds(off, _CHUNK)], sem_wh[s]),
            )

        def fire_g(s, chunk):
            for cp in g_copies(s, chunk):
                cp.start()

        def wait_g(s, chunk):
            for cp in g_copies(s, chunk):
                cp.wait()

        def fire_w(s, chunk):
            for cp in w_copies(s, chunk):
                cp.start()

        def wait_w(s, chunk):
            for cp in w_copies(s, chunk):
                cp.wait()

        fire_g(0, 0)
        fire_g(1, 1)
        for c in (0, 1):
            wait_g(c % _DEPTH, c)
            fire_w(c % _DEPTH, c)
            fire_g((c + 2) % _DEPTH, c + 2)

        @pl.loop(0, (n_chunks - 4) // _DEPTH)
        def _(m):
            for s in range(_DEPTH):
                c = m * _DEPTH + s + 2
                sl = (s + 2) % _DEPTH
                wait_g(sl, c)
                fire_w(sl, c)
                wait_w(s, c - 2)
                fire_g(s, c + 2)

        for c in (n_chunks - 2, n_chunks - 1):
            sl = c % _DEPTH
            wait_g(sl, c)
            fire_w(sl, c)
            wait_w((c - 2) % _DEPTH, c - 2)
        wait_w((n_chunks - 2) % _DEPTH, n_chunks - 2)
        wait_w((n_chunks - 1) % _DEPTH, n_chunks - 1)

    return sc_gather(ids_flat, elem_weight, hnet_weight)


def _tc_combine(hnet3, erow3, hrow3, lin_weight, tb=128, interpret=False):
    b, l, nhp = hnet3.shape
    emb = lin_weight.shape[0]

    def body(hnet_ref, e_ref, h_ref, lin_ref, out_ref):
        for t in range(tb):
            scal = lax.dot_general(
                hnet_ref[t], lin_ref[...],
                (((1,), (1,)), ((), ())),
                preferred_element_type=jnp.float32,
            )
            out_ref[t] = e_ref[t, :l, :] + h_ref[t, :l, :] * scal

    lp = erow3.shape[1]
    return pl.pallas_call(
        body,
        grid=(b // tb,),
        in_specs=[
            pl.BlockSpec((tb, l, nhp), lambda i: (i, 0, 0)),
            pl.BlockSpec((tb, lp, emb), lambda i: (i, 0, 0)),
            pl.BlockSpec((tb, lp, emb), lambda i: (i, 0, 0)),
            pl.BlockSpec((emb, nhp), lambda i: (0, 0)),
        ],
        out_specs=pl.BlockSpec((tb, l, emb), lambda i: (i, 0, 0)),
        out_shape=jax.ShapeDtypeStruct((b, l, emb), jnp.float32),
        interpret=interpret,
    )(hnet3, erow3, hrow3, lin_weight)


def kernel(input_ids, hnet_tensor, elem_weight, hnet_weight, lin_weight):
    b, l = input_ids.shape
    emb = elem_weight.shape[1]
    lp = l + (-l) % 8
    n_table = elem_weight.shape[0]
    pad_ids = (jnp.arange(b * (lp - l), dtype=jnp.int32) % n_table).reshape(
        b, lp - l)
    ids_pad = jnp.concatenate([input_ids.astype(jnp.int32), pad_ids], axis=1)
    ids_flat = ids_pad.reshape(b * lp).astype(jnp.int32)
    erow, hrow = _sc_gather_pair(ids_flat, elem_weight, hnet_weight)
    erow3 = erow.reshape(b, lp, emb)
    hrow3 = hrow.reshape(b, lp, emb)
    return _tc_combine(hnet_tensor, erow3, hrow3, lin_weight)
